# baseline (speedup 1.0000x reference)
import functools

import jax
import jax.numpy as jnp
from jax import lax
from jax.experimental import pallas as pl
from jax.experimental.pallas import tpu as pltpu
from jax.experimental.pallas import tpu_sc as plsc

N = 10000
D = 128
E = 320000

NC = 2
NS = 16
NW = NC * NS
CHUNK = 128
NCH = 80
HALF = NCH // 2
E_PAD = NW * NCH * CHUNK
PAD_DST = N + 8
ACC_ROWS = 10240
ROWS_PER_TILE = ACC_ROWS // NS


@functools.lru_cache(maxsize=None)
def _sc_kernels():
    mesh = plsc.VectorSubcoreMesh(core_axis_name="c", subcore_axis_name="s")

    @functools.partial(
        pl.kernel,
        out_type=jax.ShapeDtypeStruct((NC, ACC_ROWS, D), jnp.float32),
        mesh=mesh,
        scratch_types=[
            pltpu.VMEM((CHUNK, D), jnp.float32),
            pltpu.VMEM((CHUNK, D), jnp.float32),
            pltpu.VMEM((NCH, CHUNK), jnp.int32),
            pltpu.VMEM_SHARED((ACC_ROWS, D), jnp.float32),
        ],
    )
    def _sc_degree(dst_hbm, out_hbm, ones_v, zeros_v, didx_v, acc):
        c = lax.axis_index("c")
        s = lax.axis_index("s")
        wid = s * NC + c

        def fill(i, _):
            for t in range(D // 16):
                ones_v[i, pl.ds(t * 16, 16)] = jnp.full((16,), 1.0, jnp.float32)
                zeros_v[i, pl.ds(t * 16, 16)] = jnp.zeros((16,), jnp.float32)
            return 0

        lax.fori_loop(0, CHUNK, fill, 0)
        for k in range(ROWS_PER_TILE // CHUNK):
            pltpu.sync_copy(zeros_v, acc.at[pl.ds(s * ROWS_PER_TILE + k * CHUNK, CHUNK)])
        pltpu.sync_copy(dst_hbm.at[wid], didx_v)
        plsc.subcore_barrier()

        def body(j, _):
            pltpu.sync_copy(ones_v, acc.at[didx_v.at[j]], add=True)
            return 0

        lax.fori_loop(0, NCH, body, 0)
        plsc.subcore_barrier()
        pltpu.sync_copy(acc.at[pl.ds(s * ROWS_PER_TILE, ROWS_PER_TILE)],
                        out_hbm.at[c, pl.ds(s * ROWS_PER_TILE, ROWS_PER_TILE)])

    @functools.partial(
        pl.kernel,
        out_type=jax.ShapeDtypeStruct((NC, ACC_ROWS, D), jnp.float32),
        mesh=mesh,
        scratch_types=[
            pltpu.VMEM((CHUNK, D), jnp.float32),
            pltpu.VMEM((CHUNK, D), jnp.float32),
            pltpu.VMEM((HALF, CHUNK), jnp.int32),
            pltpu.VMEM((HALF, CHUNK), jnp.int32),
            pltpu.VMEM_SHARED((ACC_ROWS, D), jnp.float32),
            pltpu.SemaphoreType.DMA,
            pltpu.SemaphoreType.DMA,
            pltpu.SemaphoreType.DMA,
            pltpu.SemaphoreType.DMA,
        ],
    )
    def _sc_scatter(y_hbm, src_hbm, dst_hbm, out_hbm, buf_a, buf_b, sidx, didx,
                    acc, gsem_a, gsem_b, ssem_a, ssem_b):
        c = lax.axis_index("c")
        s = lax.axis_index("s")
        wid = s * NC + c

        def zrow(i, _):
            for t in range(D // 16):
                buf_a[i, pl.ds(t * 16, 16)] = jnp.zeros((16,), jnp.float32)
            return 0

        lax.fori_loop(0, CHUNK, zrow, 0)
        for k in range(ROWS_PER_TILE // CHUNK):
            pltpu.sync_copy(buf_a, acc.at[pl.ds(s * ROWS_PER_TILE + k * CHUNK, CHUNK)])
        plsc.subcore_barrier()

        for h in range(2):
            pltpu.sync_copy(src_hbm.at[wid, pl.ds(h * HALF, HALF)], sidx)
            pltpu.sync_copy(dst_hbm.at[wid, pl.ds(h * HALF, HALF)], didx)

            def body(j, _):
                pltpu.async_copy(y_hbm.at[sidx.at[j]], buf_a, gsem_a).wait()
                pltpu.sync_copy(buf_a, acc.at[didx.at[j]], add=True)
                return 0

            lax.fori_loop(0, HALF, body, 0)
        plsc.subcore_barrier()
        pltpu.sync_copy(acc.at[pl.ds(s * ROWS_PER_TILE, ROWS_PER_TILE)],
                        out_hbm.at[c, pl.ds(s * ROWS_PER_TILE, ROWS_PER_TILE)])

    return _sc_degree, _sc_scatter


_RB = 1000


def _dinv_of(degp_ref):
    deg = degp_ref[0, :, 0:1] + degp_ref[1, :, 0:1] + 1.0
    return lax.rsqrt(deg)


def _tc_first_body(x_ref, w_ref, degp_ref, y_ref):
    dinv = _dinv_of(degp_ref)
    y_ref[...] = jnp.dot(x_ref[...], w_ref[...],
                         preferred_element_type=jnp.float32) * dinv


def _tc_mid_body(s_ref, y_ref, degp_ref, b_ref, w_ref, o_ref):
    dinv = _dinv_of(degp_ref)
    t = (s_ref[0] + s_ref[1] + y_ref[...]) * dinv + b_ref[...]
    h = jnp.maximum(t, 0.0)
    o_ref[...] = jnp.dot(h, w_ref[...], preferred_element_type=jnp.float32) * dinv


def _tc_last_body(s_ref, y_ref, degp_ref, b_ref, o_ref):
    dinv = _dinv_of(degp_ref)
    o_ref[...] = (s_ref[0] + s_ref[1] + y_ref[...]) * dinv + b_ref[...]


def _tc_first(x, w, degp):
    return pl.pallas_call(
        _tc_first_body,
        grid=(N // _RB,),
        in_specs=[
            pl.BlockSpec((_RB, D), lambda i: (i, 0)),
            pl.BlockSpec((D, D), lambda i: (0, 0)),
            pl.BlockSpec((NC, _RB, D), lambda i: (0, i, 0)),
        ],
        out_specs=pl.BlockSpec((_RB, D), lambda i: (i, 0)),
        out_shape=jax.ShapeDtypeStruct((N, D), jnp.float32),
    )(x, w, degp)


def _tc_mid(s, y, degp, b, w):
    return pl.pallas_call(
        _tc_mid_body,
        grid=(N // _RB,),
        in_specs=[
            pl.BlockSpec((NC, _RB, D), lambda i: (0, i, 0)),
            pl.BlockSpec((_RB, D), lambda i: (i, 0)),
            pl.BlockSpec((NC, _RB, D), lambda i: (0, i, 0)),
            pl.BlockSpec((1, D), lambda i: (0, 0)),
            pl.BlockSpec((D, D), lambda i: (0, 0)),
        ],
        out_specs=pl.BlockSpec((_RB, D), lambda i: (i, 0)),
        out_shape=jax.ShapeDtypeStruct((N, D), jnp.float32),
    )(s, y, degp, b, w)


def _tc_last(s, y, degp, b):
    return pl.pallas_call(
        _tc_last_body,
        grid=(N // _RB,),
        in_specs=[
            pl.BlockSpec((NC, _RB, D), lambda i: (0, i, 0)),
            pl.BlockSpec((_RB, D), lambda i: (i, 0)),
            pl.BlockSpec((NC, _RB, D), lambda i: (0, i, 0)),
            pl.BlockSpec((1, D), lambda i: (0, 0)),
        ],
        out_specs=pl.BlockSpec((_RB, D), lambda i: (i, 0)),
        out_shape=jax.ShapeDtypeStruct((N, D), jnp.float32),
    )(s, y, degp, b)


def kernel(x, edge_index, W1, b1, W2, b2, W3, b3):
    src = edge_index[0].astype(jnp.int32)
    dst = edge_index[1].astype(jnp.int32)
    pad = E_PAD - E
    src3 = jnp.concatenate(
        [src, jnp.zeros((pad,), jnp.int32)]).reshape(NW, NCH, CHUNK)
    dst3 = jnp.concatenate(
        [dst, jnp.full((pad,), PAD_DST, jnp.int32)]).reshape(NW, NCH, CHUNK)

    sc_degree, sc_scatter = _sc_kernels()
    degp = sc_degree(dst3)

    b1r = b1.reshape(1, D)
    b2r = b2.reshape(1, D)
    b3r = b3.reshape(1, D)

    y1 = _tc_first(x, W1, degp)
    s1 = sc_scatter(y1, src3, dst3)
    y2 = _tc_mid(s1, y1, degp, b1r, W2)
    s2 = sc_scatter(y2, src3, dst3)
    y3 = _tc_mid(s2, y2, degp, b2r, W3)
    s3 = sc_scatter(y3, src3, dst3)
    return _tc_last(s3, y3, degp, b3r)


# speedup vs baseline: 3.1028x; 3.1028x over previous
import functools

import jax
import jax.numpy as jnp
from jax import lax
from jax.experimental import pallas as pl
from jax.experimental.pallas import tpu as pltpu
from jax.experimental.pallas import tpu_sc as plsc

N = 10000
D = 128
E = 320000

NC = 2
NS = 16
NW = NC * NS
CHUNK = 128
NCH = 80
HALF = NCH // 2
E_PAD = NW * NCH * CHUNK
PAD_DST = N + 8
ACC_ROWS = 10240
ROWS_PER_TILE = ACC_ROWS // NS


@functools.lru_cache(maxsize=None)
def _sc_kernels():
    mesh = plsc.VectorSubcoreMesh(core_axis_name="c", subcore_axis_name="s")

    @functools.partial(
        pl.kernel,
        out_type=jax.ShapeDtypeStruct((NC, ACC_ROWS, D), jnp.float32),
        mesh=mesh,
        scratch_types=[
            pltpu.VMEM((CHUNK, D), jnp.float32),
            pltpu.VMEM((CHUNK, D), jnp.float32),
            pltpu.VMEM((NCH, CHUNK), jnp.int32),
            pltpu.VMEM_SHARED((ACC_ROWS, D), jnp.float32),
        ],
    )
    def _sc_degree(dst_hbm, out_hbm, ones_v, zeros_v, didx_v, acc):
        c = lax.axis_index("c")
        s = lax.axis_index("s")
        wid = s * NC + c

        def fill(i, _):
            for t in range(D // 16):
                ones_v[i, pl.ds(t * 16, 16)] = jnp.full((16,), 1.0, jnp.float32)
                zeros_v[i, pl.ds(t * 16, 16)] = jnp.zeros((16,), jnp.float32)
            return 0

        lax.fori_loop(0, CHUNK, fill, 0)
        for k in range(ROWS_PER_TILE // CHUNK):
            pltpu.sync_copy(zeros_v, acc.at[pl.ds(s * ROWS_PER_TILE + k * CHUNK, CHUNK)])
        pltpu.sync_copy(dst_hbm.at[wid], didx_v)
        plsc.subcore_barrier()

        def body(j, _):
            pltpu.sync_copy(ones_v, acc.at[didx_v.at[j]], add=True)
            return 0

        lax.fori_loop(0, NCH, body, 0)
        plsc.subcore_barrier()
        pltpu.sync_copy(acc.at[pl.ds(s * ROWS_PER_TILE, ROWS_PER_TILE)],
                        out_hbm.at[c, pl.ds(s * ROWS_PER_TILE, ROWS_PER_TILE)])

    @functools.partial(
        pl.kernel,
        out_type=jax.ShapeDtypeStruct((NC, ACC_ROWS, D), jnp.float32),
        mesh=mesh,
        scratch_types=[
            pltpu.VMEM((CHUNK, D), jnp.float32),
            pltpu.VMEM((CHUNK, D), jnp.float32),
            pltpu.VMEM((HALF, CHUNK), jnp.int32),
            pltpu.VMEM((HALF, CHUNK), jnp.int32),
            pltpu.VMEM_SHARED((ACC_ROWS, D), jnp.float32),
            pltpu.SemaphoreType.DMA,
            pltpu.SemaphoreType.DMA,
            pltpu.SemaphoreType.DMA,
            pltpu.SemaphoreType.DMA,
        ],
    )
    def _sc_scatter(y_hbm, src_hbm, dst_hbm, out_hbm, buf_a, buf_b, sidx, didx,
                    acc, gsem_a, gsem_b, ssem_a, ssem_b):
        c = lax.axis_index("c")
        s = lax.axis_index("s")
        wid = s * NC + c

        def zrow(i, _):
            for t in range(D // 16):
                buf_a[i, pl.ds(t * 16, 16)] = jnp.zeros((16,), jnp.float32)
            return 0

        lax.fori_loop(0, CHUNK, zrow, 0)
        for k in range(ROWS_PER_TILE // CHUNK):
            pltpu.sync_copy(buf_a, acc.at[pl.ds(s * ROWS_PER_TILE + k * CHUNK, CHUNK)])
        plsc.subcore_barrier()

        for h in range(2):
            pltpu.sync_copy(src_hbm.at[wid, pl.ds(h * HALF, HALF)], sidx)
            pltpu.sync_copy(dst_hbm.at[wid, pl.ds(h * HALF, HALF)], didx)
            pltpu.async_copy(y_hbm.at[sidx.at[0]], buf_a, gsem_a)

            def pair(p, _):
                j = 2 * p
                pltpu.make_async_copy(y_hbm.at[sidx.at[j]], buf_a, gsem_a).wait()
                pltpu.async_copy(y_hbm.at[sidx.at[j + 1]], buf_b, gsem_b)
                pltpu.sync_copy(buf_a, acc.at[didx.at[j]], add=True)
                pltpu.make_async_copy(y_hbm.at[sidx.at[j + 1]], buf_b, gsem_b).wait()

                @pl.when(p < HALF // 2 - 1)
                def _():
                    pltpu.async_copy(y_hbm.at[sidx.at[j + 2]], buf_a, gsem_a)

                pltpu.sync_copy(buf_b, acc.at[didx.at[j + 1]], add=True)
                return 0

            lax.fori_loop(0, HALF // 2, pair, 0)
        plsc.subcore_barrier()
        pltpu.sync_copy(acc.at[pl.ds(s * ROWS_PER_TILE, ROWS_PER_TILE)],
                        out_hbm.at[c, pl.ds(s * ROWS_PER_TILE, ROWS_PER_TILE)])

    return _sc_degree, _sc_scatter


_RB = 1000


def _dinv_of(degp_ref):
    deg = degp_ref[0, :, 0:1] + degp_ref[1, :, 0:1] + 1.0
    return lax.rsqrt(deg)


def _tc_first_body(x_ref, w_ref, degp_ref, y_ref):
    dinv = _dinv_of(degp_ref)
    y_ref[...] = jnp.dot(x_ref[...], w_ref[...],
                         preferred_element_type=jnp.float32) * dinv


def _tc_mid_body(s_ref, y_ref, degp_ref, b_ref, w_ref, o_ref):
    dinv = _dinv_of(degp_ref)
    t = (s_ref[0] + s_ref[1] + y_ref[...]) * dinv + b_ref[...]
    h = jnp.maximum(t, 0.0)
    o_ref[...] = jnp.dot(h, w_ref[...], preferred_element_type=jnp.float32) * dinv


def _tc_last_body(s_ref, y_ref, degp_ref, b_ref, o_ref):
    dinv = _dinv_of(degp_ref)
    o_ref[...] = (s_ref[0] + s_ref[1] + y_ref[...]) * dinv + b_ref[...]


def _tc_first(x, w, degp):
    return pl.pallas_call(
        _tc_first_body,
        grid=(N // _RB,),
        in_specs=[
            pl.BlockSpec((_RB, D), lambda i: (i, 0)),
            pl.BlockSpec((D, D), lambda i: (0, 0)),
            pl.BlockSpec((NC, _RB, D), lambda i: (0, i, 0)),
        ],
        out_specs=pl.BlockSpec((_RB, D), lambda i: (i, 0)),
        out_shape=jax.ShapeDtypeStruct((N, D), jnp.float32),
    )(x, w, degp)


def _tc_mid(s, y, degp, b, w):
    return pl.pallas_call(
        _tc_mid_body,
        grid=(N // _RB,),
        in_specs=[
            pl.BlockSpec((NC, _RB, D), lambda i: (0, i, 0)),
            pl.BlockSpec((_RB, D), lambda i: (i, 0)),
            pl.BlockSpec((NC, _RB, D), lambda i: (0, i, 0)),
            pl.BlockSpec((1, D), lambda i: (0, 0)),
            pl.BlockSpec((D, D), lambda i: (0, 0)),
        ],
        out_specs=pl.BlockSpec((_RB, D), lambda i: (i, 0)),
        out_shape=jax.ShapeDtypeStruct((N, D), jnp.float32),
    )(s, y, degp, b, w)


def _tc_last(s, y, degp, b):
    return pl.pallas_call(
        _tc_last_body,
        grid=(N // _RB,),
        in_specs=[
            pl.BlockSpec((NC, _RB, D), lambda i: (0, i, 0)),
            pl.BlockSpec((_RB, D), lambda i: (i, 0)),
            pl.BlockSpec((NC, _RB, D), lambda i: (0, i, 0)),
            pl.BlockSpec((1, D), lambda i: (0, 0)),
        ],
        out_specs=pl.BlockSpec((_RB, D), lambda i: (i, 0)),
        out_shape=jax.ShapeDtypeStruct((N, D), jnp.float32),
    )(s, y, degp, b)


def kernel(x, edge_index, W1, b1, W2, b2, W3, b3):
    src = edge_index[0].astype(jnp.int32)
    dst = edge_index[1].astype(jnp.int32)
    pad = E_PAD - E
    padi = jnp.arange(pad, dtype=jnp.int32)
    src3 = jnp.concatenate([src, padi % N]).reshape(NW, NCH, CHUNK)
    dst3 = jnp.concatenate(
        [dst, N + padi % (ACC_ROWS - N)]).reshape(NW, NCH, CHUNK)

    sc_degree, sc_scatter = _sc_kernels()
    degp = sc_degree(dst3)

    b1r = b1.reshape(1, D)
    b2r = b2.reshape(1, D)
    b3r = b3.reshape(1, D)

    y1 = _tc_first(x, W1, degp)
    s1 = sc_scatter(y1, src3, dst3)
    y2 = _tc_mid(s1, y1, degp, b1r, W2)
    s2 = sc_scatter(y2, src3, dst3)
    y3 = _tc_mid(s2, y2, degp, b2r, W3)
    s3 = sc_scatter(y3, src3, dst3)
    return _tc_last(s3, y3, degp, b3r)
